# Initial kernel scaffold; baseline (speedup 1.0000x reference)
import jax
import jax.numpy as jnp
from jax.experimental import pallas as pl


def kernel(x, W1, W2):
    raise NotImplementedError("write your pallas kernel here")


# speedup vs baseline: 1.0010x; 1.0010x over previous
import jax
import jax.numpy as jnp
from jax.experimental import pallas as pl
from jax.experimental.pallas import tpu as pltpu

_S_BLK = 512


def _router_kernel(x_ref, w1_ref, w2_ref, out_ref, acc_ref):
    i = pl.program_id(0)
    d_model = x_ref.shape[-1]
    m = jnp.sum(x_ref[...], axis=-1) * (1.0 / d_model)
    mt = m.T
    part = jax.lax.dot_general(
        w1_ref[...], mt, (((1,), (0,)), ((), ())),
        preferred_element_type=jnp.float32)

    @pl.when(i == 0)
    def _():
        acc_ref[...] = part

    @pl.when(i > 0)
    def _():
        acc_ref[...] = acc_ref[...] + part

    @pl.when(i == pl.num_programs(0) - 1)
    def _():
        h = jnp.maximum(acc_ref[...], 0.0)
        o = jax.lax.dot_general(
            w2_ref[...], h, (((1,), (0,)), ((), ())),
            preferred_element_type=jnp.float32)
        out_ref[...] = o.T


def kernel(x, W1, W2):
    b, seq_len, d_model = x.shape
    router_size = W1.shape[0]
    num_experts = W2.shape[0]
    grid = (seq_len // _S_BLK,)
    return pl.pallas_call(
        _router_kernel,
        grid=grid,
        in_specs=[
            pl.BlockSpec((b, _S_BLK, d_model), lambda i: (0, i, 0)),
            pl.BlockSpec((router_size, _S_BLK), lambda i: (0, i)),
            pl.BlockSpec((num_experts, router_size), lambda i: (0, 0)),
        ],
        out_specs=pl.BlockSpec((b, num_experts), lambda i: (0, 0)),
        out_shape=jax.ShapeDtypeStruct((b, num_experts), jnp.float32),
        scratch_shapes=[pltpu.VMEM((router_size, b), jnp.float32)],
        compiler_params=pltpu.CompilerParams(
            dimension_semantics=("arbitrary",),
        ),
    )(x, W1, W2)
